# baseline (speedup 1.0000x reference)
import functools

import jax
import jax.numpy as jnp
from jax import lax
from jax.experimental import pallas as pl
from jax.experimental.pallas import tpu as pltpu
from jax.experimental.pallas import tpu_sc as plsc

BATCH = 16384
UD = 32
MD = 32
H1 = 128
H2 = 64

NUM_WORKERS = 32
CHUNK = 128
ROWS_PER_W = BATCH // NUM_WORKERS
CHUNKS_PER_W = ROWS_PER_W // CHUNK
IDX_ROWS = BATCH // CHUNK


def _sc_gather(user_table, movie_table, uidx2d, midx2d):
    mesh = plsc.VectorSubcoreMesh(core_axis_name="c", subcore_axis_name="s")

    @functools.partial(
        pl.kernel,
        mesh=mesh,
        compiler_params=pltpu.CompilerParams(use_tc_tiling_on_sc=False),
        out_type=[
            jax.ShapeDtypeStruct((BATCH, UD), jnp.float32),
            jax.ShapeDtypeStruct((BATCH, MD), jnp.float32),
        ],
        scratch_types=[
            pltpu.VMEM((CHUNKS_PER_W, CHUNK), jnp.int32),
            pltpu.VMEM((CHUNKS_PER_W, CHUNK), jnp.int32),
            pltpu.VMEM((ROWS_PER_W, UD), jnp.float32),
            pltpu.VMEM((ROWS_PER_W, MD), jnp.float32),
            pltpu.SemaphoreType.DMA,
        ],
    )
    def k(ut_hbm, mt_hbm, ui_hbm, mi_hbm, uout_hbm, mout_hbm,
          ui_v, mi_v, urows_v, mrows_v, sem):
        wid = lax.axis_index("s") * 2 + lax.axis_index("c")
        idx_row0 = wid * CHUNKS_PER_W
        pltpu.sync_copy(ui_hbm.at[pl.ds(idx_row0, CHUNKS_PER_W)], ui_v)
        pltpu.sync_copy(mi_hbm.at[pl.ds(idx_row0, CHUNKS_PER_W)], mi_v)
        copies = []
        for j in range(CHUNKS_PER_W):
            copies.append(pltpu.async_copy(
                ut_hbm.at[ui_v.at[j]],
                urows_v.at[pl.ds(j * CHUNK, CHUNK)], sem))
        for j in range(CHUNKS_PER_W):
            copies.append(pltpu.async_copy(
                mt_hbm.at[mi_v.at[j]],
                mrows_v.at[pl.ds(j * CHUNK, CHUNK)], sem))
        for c in copies:
            c.wait()
        base = wid * ROWS_PER_W
        pltpu.sync_copy(urows_v, uout_hbm.at[pl.ds(base, ROWS_PER_W)])
        pltpu.sync_copy(mrows_v, mout_hbm.at[pl.ds(base, ROWS_PER_W)])

    return k(user_table, movie_table, uidx2d, midx2d)


BLK = 2048
GRID = BATCH // BLK


def _mlp_body(ue_ref, me_ref, age_ref, rat_ref, w1u_ref, w1m_ref, war_ref,
              b1_ref, w2_ref, b2_ref, w3_ref, b3_ref, out_ref):
    hp = jax.lax.Precision.HIGHEST
    h = jnp.dot(ue_ref[...], w1u_ref[...], precision=hp,
                preferred_element_type=jnp.float32)
    h = h + jnp.dot(me_ref[...], w1m_ref[...], precision=hp,
                    preferred_element_type=jnp.float32)
    war = war_ref[...]
    h = h + age_ref[...] * war[0:1, :] + rat_ref[...] * war[1:2, :]
    h = jnp.maximum(h + b1_ref[...], 0.0)
    h = jnp.dot(h, w2_ref[...], precision=hp, preferred_element_type=jnp.float32)
    h = jnp.maximum(h + b2_ref[...], 0.0)
    logit = jnp.sum(h * w3_ref[...], axis=1, keepdims=True) + b3_ref[...]
    out_ref[...] = jax.nn.sigmoid(logit)


def _mlp(ue, me, ages2d, rats2d, W1u, W1m, War, b1, W2, b2, w3row, b3):
    full = lambda i: (0, 0)
    out = pl.pallas_call(
        _mlp_body,
        grid=(GRID,),
        in_specs=[
            pl.BlockSpec((BLK, UD), lambda i: (i, 0)),
            pl.BlockSpec((BLK, MD), lambda i: (i, 0)),
            pl.BlockSpec((BLK, 1), lambda i: (i, 0)),
            pl.BlockSpec((BLK, 1), lambda i: (i, 0)),
            pl.BlockSpec((UD, H1), full),
            pl.BlockSpec((MD, H1), full),
            pl.BlockSpec((2, H1), full),
            pl.BlockSpec((1, H1), full),
            pl.BlockSpec((H1, H2), full),
            pl.BlockSpec((1, H2), full),
            pl.BlockSpec((1, H2), full),
            pl.BlockSpec((1, 1), full),
        ],
        out_specs=pl.BlockSpec((BLK, 1), lambda i: (i, 0)),
        out_shape=jax.ShapeDtypeStruct((BATCH, 1), jnp.float32),
    )(ue, me, ages2d, rats2d, W1u, W1m, War, b1, W2, b2, w3row, b3)
    return out


def kernel(user_ids, movie_ids, user_ages, movie_ratings,
           user_table, movie_table, W1, b1, W2, b2, W3, b3):
    uidx2d = user_ids.astype(jnp.int32).reshape(IDX_ROWS, CHUNK)
    midx2d = movie_ids.astype(jnp.int32).reshape(IDX_ROWS, CHUNK)
    ue, me = _sc_gather(user_table, movie_table, uidx2d, midx2d)
    W1u = W1[0:UD, :]
    W1m = W1[UD:UD + MD, :]
    War = W1[UD + MD:UD + MD + 2, :]
    out = _mlp(ue, me,
               user_ages.astype(jnp.float32)[:, None],
               movie_ratings.astype(jnp.float32)[:, None],
               W1u, W1m, War, b1[None, :], W2, b2[None, :],
               W3.T, b3[None, :])
    return out[:, 0]
